# baseline (speedup 1.0000x reference)
import functools

import jax
import jax.numpy as jnp
import numpy as np
from jax.experimental import pallas as pl
from jax.experimental.pallas import tpu as pltpu

E = 8
TOP_K = 2
D = 13
H = 10
EH = E * H

_KS0 = np.uint32(0)
_KS1 = np.uint32(1)
_KS2 = np.uint32(0 ^ 1 ^ 0x1BD11BDA)
_ROT0 = (13, 15, 26, 6)
_ROT1 = (17, 29, 16, 24)

_MINVAL = np.float32(np.nextafter(np.float32(-1), np.float32(0)))
_SPAN = np.float32(np.float32(1.0) - _MINVAL)
_SQRT2 = np.float32(np.sqrt(2.0))

_ERFINV_SMALL = (2.81022636e-08, 3.43273939e-07, -3.5233877e-06,
                 -4.39150654e-06, 0.00021858087, -0.00125372503,
                 -0.00417768164, 0.246640727, 1.50140941)
_ERFINV_BIG = (-0.000200214257, 0.000100950558, 0.00134934322,
               -0.00367342844, 0.00573950773, -0.0076224613,
               0.00943887047, 1.00167406, 2.83297682)


def _rotl(v, r):
    return jax.lax.shift_left(v, np.uint32(r)) | jax.lax.shift_right_logical(
        v, np.uint32(32 - r))


def _threefry2x32(c0, c1):
    x0 = c0 + _KS0
    x1 = c1 + _KS1
    ks = (_KS0, _KS1, _KS2)
    for i in range(5):
        for r in (_ROT0 if i % 2 == 0 else _ROT1):
            x0 = x0 + x1
            x1 = _rotl(x1, r) ^ x0
        x0 = x0 + ks[(i + 1) % 3]
        x1 = x1 + ks[(i + 2) % 3] + np.uint32(i + 1)
    return x0, x1


def _bits_to_normal(bits):
    fb = jax.lax.shift_right_logical(bits, np.uint32(9)) | np.uint32(0x3F800000)
    u01 = jax.lax.bitcast_convert_type(fb, jnp.float32) - 1.0
    u = jnp.maximum(_MINVAL, u01 * _SPAN + _MINVAL)
    w = -jnp.log1p(-u * u)
    lt = w < 5.0
    ws = w - 2.5
    wb = jnp.sqrt(w) - 3.0
    ps = jnp.float32(_ERFINV_SMALL[0])
    pb = jnp.float32(_ERFINV_BIG[0])
    for cs, cb in zip(_ERFINV_SMALL[1:], _ERFINV_BIG[1:]):
        ps = ps * ws + cs
        pb = pb * wb + cb
    p = jnp.where(lt, ps, pb)
    return _SQRT2 * (p * u)


def _moe_block(x_ref, wall_ref, ball_ref, rept_ref, w2aug_ref, out_ref,
               *, block):
    xt = x_ref[...].T
    pt = jnp.dot(wall_ref[...], xt, preferred_element_type=jnp.float32)
    pt = pt + ball_ref[...]
    ht = jnp.maximum(pt[0:EH, :], 0.0)
    lgt = pt[EH:EH + E, :]
    nlt = pt[EH + E:EH + 2 * E, :]

    n0 = (pl.program_id(0) * block).astype(jnp.uint32)
    lane = jax.lax.broadcasted_iota(jnp.uint32, (E, block), 1)
    sub = jax.lax.broadcasted_iota(jnp.uint32, (E, block), 0)
    idx = (n0 + lane) * np.uint32(E) + sub
    o0, o1 = _threefry2x32(jnp.zeros_like(idx), idx)
    noise = _bits_to_normal(o0 ^ o1)

    noisy = lgt + noise * jax.nn.softplus(nlt)

    eidx = jax.lax.broadcasted_iota(jnp.int32, (E, block), 0)
    m1 = jnp.max(noisy, axis=0, keepdims=True)
    i1 = jnp.min(jnp.where(noisy == m1, eidx, E), axis=0, keepdims=True)
    mask1 = eidx == i1
    rest = jnp.where(mask1, -jnp.inf, noisy)
    m2 = jnp.max(rest, axis=0, keepdims=True)
    i2 = jnp.min(jnp.where(rest == m2, eidx, E), axis=0, keepdims=True)
    sel = mask1 | (eidx == i2)

    wts = jnp.where(sel, jnp.exp(noisy - m1), 0.0)
    router = wts / jnp.sum(wts, axis=0, keepdims=True)

    rep = jnp.dot(rept_ref[...], router, preferred_element_type=jnp.float32)
    cat = jnp.concatenate([ht * rep, router], axis=0)
    out = jnp.dot(w2aug_ref[...], cat,
                  preferred_element_type=jnp.float32)
    out_ref[...] = out.T


@functools.partial(jax.jit, static_argnames=("block",))
def _moe(x, Wg, bg, Wn, bn, W1, b1, W2, b2, block):
    n = x.shape[0]
    wall = jnp.concatenate([W1.reshape(EH, D), Wg, Wn], axis=0)
    ball = jnp.concatenate([b1.reshape(EH), bg, bn]).reshape(EH + 2 * E, 1)
    rept = jnp.kron(jnp.eye(E, dtype=x.dtype), jnp.ones((H, 1), dtype=x.dtype))
    w2aug = jnp.concatenate([W2.transpose(0, 2, 1).reshape(EH, D).T, b2.T],
                            axis=1)

    grid = (n // block,)
    full = lambda r, c: pl.BlockSpec((r, c), lambda i: (0, 0))
    return pl.pallas_call(
        functools.partial(_moe_block, block=block),
        grid=grid,
        in_specs=[
            pl.BlockSpec((block, D), lambda i: (i, 0)),
            full(EH + 2 * E, D), full(EH + 2 * E, 1),
            full(EH, E), full(D, EH + E),
        ],
        out_specs=pl.BlockSpec((block, D), lambda i: (i, 0)),
        out_shape=jax.ShapeDtypeStruct((n, D), x.dtype),
        compiler_params=pltpu.CompilerParams(
            dimension_semantics=("arbitrary",)),
    )(x, wall, ball, rept, w2aug)


def kernel(x, Wg, bg, Wn, bn, W1, b1, W2, b2):
    return _moe(x, Wg, bg, Wn, bn, W1, b1, W2, b2, block=4096)


# speedup vs baseline: 1.3286x; 1.3286x over previous
import functools

import jax
import jax.numpy as jnp
import numpy as np
from jax.experimental import pallas as pl
from jax.experimental.pallas import tpu as pltpu

E = 8
TOP_K = 2
D = 13
H = 10
EH = E * H

_KS0 = np.uint32(0)
_KS1 = np.uint32(1)
_KS2 = np.uint32(0 ^ 1 ^ 0x1BD11BDA)
_ROT0 = (13, 15, 26, 6)
_ROT1 = (17, 29, 16, 24)

_MINVAL = np.float32(np.nextafter(np.float32(-1), np.float32(0)))
_SPAN = np.float32(np.float32(1.0) - _MINVAL)
_SQRT2 = np.float32(np.sqrt(2.0))

_ERFINV_SMALL = (2.81022636e-08, 3.43273939e-07, -3.5233877e-06,
                 -4.39150654e-06, 0.00021858087, -0.00125372503,
                 -0.00417768164, 0.246640727, 1.50140941)
_ERFINV_BIG = (-0.000200214257, 0.000100950558, 0.00134934322,
               -0.00367342844, 0.00573950773, -0.0076224613,
               0.00943887047, 1.00167406, 2.83297682)


def _rotl(v, r):
    return jax.lax.shift_left(v, np.uint32(r)) | jax.lax.shift_right_logical(
        v, np.uint32(32 - r))


def _threefry2x32(c0, c1):
    x0 = c0 + _KS0
    x1 = c1 + _KS1
    ks = (_KS0, _KS1, _KS2)
    for i in range(5):
        for r in (_ROT0 if i % 2 == 0 else _ROT1):
            x0 = x0 + x1
            x1 = _rotl(x1, r) ^ x0
        x0 = x0 + ks[(i + 1) % 3]
        x1 = x1 + ks[(i + 2) % 3] + np.uint32(i + 1)
    return x0, x1


def _bits_to_normal(bits):
    fb = jax.lax.shift_right_logical(bits, np.uint32(9)) | np.uint32(0x3F800000)
    u01 = jax.lax.bitcast_convert_type(fb, jnp.float32) - 1.0
    u = jnp.maximum(_MINVAL, u01 * _SPAN + _MINVAL)
    w = -jnp.log1p(-u * u)
    lt = w < 5.0
    ws = w - 2.5
    wb = jnp.sqrt(w) - 3.0
    ps = jnp.float32(_ERFINV_SMALL[0])
    pb = jnp.float32(_ERFINV_BIG[0])
    for cs, cb in zip(_ERFINV_SMALL[1:], _ERFINV_BIG[1:]):
        ps = ps * ws + cs
        pb = pb * wb + cb
    p = jnp.where(lt, ps, pb)
    return _SQRT2 * (p * u)


def _moe_block(xt_ref, wall_ref, ball_ref, rept_ref, w2aug_ref, out_ref,
               *, block):
    xt = xt_ref[...]
    pt = jnp.dot(wall_ref[...], xt, preferred_element_type=jnp.float32)
    pt = pt + ball_ref[...]
    ht = jnp.maximum(pt[0:EH, :], 0.0)
    lgt = pt[EH:EH + E, :]
    nlt = pt[EH + E:EH + 2 * E, :]

    n0 = (pl.program_id(0) * block).astype(jnp.uint32)
    lane = jax.lax.broadcasted_iota(jnp.uint32, (E, block), 1)
    sub = jax.lax.broadcasted_iota(jnp.uint32, (E, block), 0)
    idx = (n0 + lane) * np.uint32(E) + sub
    o0, o1 = _threefry2x32(jnp.zeros_like(idx), idx)
    noise = _bits_to_normal(o0 ^ o1)

    noisy = lgt + noise * jax.nn.softplus(nlt)

    eidx = jax.lax.broadcasted_iota(jnp.int32, (E, block), 0)
    m1 = jnp.max(noisy, axis=0, keepdims=True)
    i1 = jnp.min(jnp.where(noisy == m1, eidx, E), axis=0, keepdims=True)
    mask1 = eidx == i1
    rest = jnp.where(mask1, -jnp.inf, noisy)
    m2 = jnp.max(rest, axis=0, keepdims=True)
    i2 = jnp.min(jnp.where(rest == m2, eidx, E), axis=0, keepdims=True)
    sel = mask1 | (eidx == i2)

    wts = jnp.where(sel, jnp.exp(noisy - m1), 0.0)
    router = wts / jnp.sum(wts, axis=0, keepdims=True)

    rep = jnp.dot(rept_ref[...], router, preferred_element_type=jnp.float32)
    cat = jnp.concatenate([ht * rep, router], axis=0)
    out = jnp.dot(w2aug_ref[...], cat,
                  preferred_element_type=jnp.float32)
    out_ref[...] = out.T


@functools.partial(jax.jit, static_argnames=("block",))
def _moe(x, Wg, bg, Wn, bn, W1, b1, W2, b2, block):
    n = x.shape[0]
    xt = x.T
    wall = jnp.concatenate([W1.reshape(EH, D), Wg, Wn], axis=0)
    ball = jnp.concatenate([b1.reshape(EH), bg, bn]).reshape(EH + 2 * E, 1)
    rept = jnp.kron(jnp.eye(E, dtype=x.dtype), jnp.ones((H, 1), dtype=x.dtype))
    w2aug = jnp.concatenate([W2.transpose(0, 2, 1).reshape(EH, D).T, b2.T],
                            axis=1)

    grid = (n // block,)
    full = lambda r, c: pl.BlockSpec((r, c), lambda i: (0, 0))
    return pl.pallas_call(
        functools.partial(_moe_block, block=block),
        grid=grid,
        in_specs=[
            pl.BlockSpec((D, block), lambda i: (0, i)),
            full(EH + 2 * E, D), full(EH + 2 * E, 1),
            full(EH, E), full(D, EH + E),
        ],
        out_specs=pl.BlockSpec((block, D), lambda i: (i, 0)),
        out_shape=jax.ShapeDtypeStruct((n, D), x.dtype),
        compiler_params=pltpu.CompilerParams(
            dimension_semantics=("arbitrary",)),
    )(xt, wall, ball, rept, w2aug)


def kernel(x, Wg, bg, Wn, bn, W1, b1, W2, b2):
    return _moe(x, Wg, bg, Wn, bn, W1, b1, W2, b2, block=4096)


# speedup vs baseline: 2.2819x; 1.7175x over previous
import functools

import jax
import jax.numpy as jnp
import numpy as np
from jax.experimental import pallas as pl
from jax.experimental.pallas import tpu as pltpu

E = 8
TOP_K = 2
D = 13
H = 10
EH = E * H

_KS0 = np.uint32(0)
_KS1 = np.uint32(1)
_KS2 = np.uint32(0 ^ 1 ^ 0x1BD11BDA)
_ROT0 = (13, 15, 26, 6)
_ROT1 = (17, 29, 16, 24)

_MINVAL = np.float32(np.nextafter(np.float32(-1), np.float32(0)))
_SPAN = np.float32(np.float32(1.0) - _MINVAL)
_SQRT2 = np.float32(np.sqrt(2.0))

_ERFINV_SMALL = (2.81022636e-08, 3.43273939e-07, -3.5233877e-06,
                 -4.39150654e-06, 0.00021858087, -0.00125372503,
                 -0.00417768164, 0.246640727, 1.50140941)
_ERFINV_BIG = (-0.000200214257, 0.000100950558, 0.00134934322,
               -0.00367342844, 0.00573950773, -0.0076224613,
               0.00943887047, 1.00167406, 2.83297682)


def _rotl(v, r):
    return jax.lax.shift_left(v, np.uint32(r)) | jax.lax.shift_right_logical(
        v, np.uint32(32 - r))


def _threefry2x32(c0, c1):
    x0 = c0 + _KS0
    x1 = c1 + _KS1
    ks = (_KS0, _KS1, _KS2)
    for i in range(5):
        for r in (_ROT0 if i % 2 == 0 else _ROT1):
            x0 = x0 + x1
            x1 = _rotl(x1, r) ^ x0
        x0 = x0 + ks[(i + 1) % 3]
        x1 = x1 + ks[(i + 2) % 3] + np.uint32(i + 1)
    return x0, x1


def _bits_to_normal(bits):
    fb = jax.lax.shift_right_logical(bits, np.uint32(9)) | np.uint32(0x3F800000)
    u01 = jax.lax.bitcast_convert_type(fb, jnp.float32) - 1.0
    u = jnp.maximum(_MINVAL, u01 * _SPAN + _MINVAL)
    w = -jnp.log1p(-u * u)
    lt = w < 5.0
    ws = w - 2.5
    wb = jnp.sqrt(w) - 3.0
    ps = jnp.float32(_ERFINV_SMALL[0])
    pb = jnp.float32(_ERFINV_BIG[0])
    for cs, cb in zip(_ERFINV_SMALL[1:], _ERFINV_BIG[1:]):
        ps = ps * ws + cs
        pb = pb * wb + cb
    p = jnp.where(lt, ps, pb)
    return _SQRT2 * (p * u)


def _moe_block(xt_ref, wall_ref, ball_ref, rept_ref, w2aug_ref, out_ref,
               *, block):
    xt = xt_ref[...]
    pt = jnp.dot(wall_ref[...], xt, preferred_element_type=jnp.float32)
    pt = pt + ball_ref[...]
    ht = jnp.maximum(pt[0:EH, :], 0.0)
    lgt = pt[EH:EH + E, :]
    nlt = pt[EH + E:EH + 2 * E, :]

    n0 = (pl.program_id(0) * block).astype(jnp.uint32)
    lane = jax.lax.broadcasted_iota(jnp.uint32, (E, block), 1)
    sub = jax.lax.broadcasted_iota(jnp.uint32, (E, block), 0)
    idx = (n0 + lane) * np.uint32(E) + sub
    o0, o1 = _threefry2x32(jnp.zeros_like(idx), idx)
    noise = _bits_to_normal(o0 ^ o1)

    noisy = lgt + noise * jax.nn.softplus(nlt)

    eidx = jax.lax.broadcasted_iota(jnp.int32, (E, block), 0)
    m1 = jnp.max(noisy, axis=0, keepdims=True)
    i1 = jnp.min(jnp.where(noisy == m1, eidx, E), axis=0, keepdims=True)
    mask1 = eidx == i1
    rest = jnp.where(mask1, -jnp.inf, noisy)
    m2 = jnp.max(rest, axis=0, keepdims=True)
    i2 = jnp.min(jnp.where(rest == m2, eidx, E), axis=0, keepdims=True)
    sel = mask1 | (eidx == i2)

    wts = jnp.where(sel, jnp.exp(noisy - m1), 0.0)
    router = wts / jnp.sum(wts, axis=0, keepdims=True)

    rep = jnp.dot(rept_ref[...], router, preferred_element_type=jnp.float32)
    cat = jnp.concatenate([ht * rep, router], axis=0)
    out_ref[...] = jnp.dot(w2aug_ref[...], cat,
                           preferred_element_type=jnp.float32)


@functools.partial(jax.jit, static_argnames=("block",))
def _moe(x, Wg, bg, Wn, bn, W1, b1, W2, b2, block):
    n = x.shape[0]
    xt = x.T
    wall = jnp.concatenate([W1.reshape(EH, D), Wg, Wn], axis=0)
    ball = jnp.concatenate([b1.reshape(EH), bg, bn]).reshape(EH + 2 * E, 1)
    rept = jnp.kron(jnp.eye(E, dtype=x.dtype), jnp.ones((H, 1), dtype=x.dtype))
    w2aug = jnp.concatenate([W2.transpose(0, 2, 1).reshape(EH, D).T, b2.T],
                            axis=1)

    grid = (n // block,)
    full = lambda r, c: pl.BlockSpec((r, c), lambda i: (0, 0))
    out_t = pl.pallas_call(
        functools.partial(_moe_block, block=block),
        grid=grid,
        in_specs=[
            pl.BlockSpec((D, block), lambda i: (0, i)),
            full(EH + 2 * E, D), full(EH + 2 * E, 1),
            full(EH, E), full(D, EH + E),
        ],
        out_specs=pl.BlockSpec((D, block), lambda i: (0, i)),
        out_shape=jax.ShapeDtypeStruct((D, n), x.dtype),
        compiler_params=pltpu.CompilerParams(
            dimension_semantics=("arbitrary",)),
    )(xt, wall, ball, rept, w2aug)
    return out_t.T


def kernel(x, Wg, bg, Wn, bn, W1, b1, W2, b2):
    return _moe(x, Wg, bg, Wn, bn, W1, b1, W2, b2, block=8192)
